# baseline (speedup 1.0000x reference)
import functools

import jax
import jax.numpy as jnp
from jax import lax
from jax.experimental import pallas as pl
from jax.experimental.pallas import tpu as pltpu
from jax.experimental.pallas import tpu_sc as plsc

_K = 1024
_D = 64
_B = 1024
_COMMITMENT_COST = 1.0


def _vq_tc(xt_ref, et_ref, qt_ref, idx_ref, loss_ref):
    xt = xt_ref[:].reshape(_D, _B)
    et = et_ref[:]
    en = jnp.sum(et * et, axis=0)
    prod = jax.lax.dot_general(
        et, xt, (((0,), (0,)), ((), ())), preferred_element_type=jnp.float32
    )
    dist = en[:, None] - 2.0 * prod
    idx = jnp.argmin(dist, axis=0).astype(jnp.int32)
    idx_ref[:] = idx
    iota = jax.lax.broadcasted_iota(jnp.int32, (_K, _B), 0)
    oh = (iota == idx[None, :]).astype(jnp.bfloat16)
    qt = jax.lax.dot_general(
        et.astype(jnp.bfloat16), oh, (((1,), (0,)), ((), ())),
        preferred_element_type=jnp.float32,
    )
    qt_ref[:] = qt

    i = pl.program_id(0)

    @pl.when(i == 0)
    def _init():
        loss_ref[0, 0] = 0.0

    loss_ref[0, 0] += jnp.sum((xt - qt) ** 2)


def kernel(inputs, embedding):
    g, bper, _ = inputs.shape
    n = g * bper
    xt3 = jnp.transpose(inputs, (0, 2, 1))
    et = embedding.T
    qt, idx, losssum = pl.pallas_call(
        _vq_tc,
        grid=(n // _B,),
        in_specs=[
            pl.BlockSpec((_B // bper, _D, bper), lambda i: (i, 0, 0)),
            pl.BlockSpec((_D, _K), lambda i: (0, 0)),
        ],
        out_specs=[
            pl.BlockSpec((_D, _B), lambda i: (0, i)),
            pl.BlockSpec((_B,), lambda i: (i,)),
            pl.BlockSpec((1, 1), lambda i: (0, 0), memory_space=pltpu.SMEM),
        ],
        out_shape=[
            jax.ShapeDtypeStruct((_D, n), jnp.float32),
            jax.ShapeDtypeStruct((n,), jnp.int32),
            jax.ShapeDtypeStruct((1, 1), jnp.float32),
        ],
    )(xt3, et)
    loss = _COMMITMENT_COST * (losssum[0, 0] / (n * _D))
    q = _sc_gather(embedding, idx)
    return q, loss, idx


_NC = 2
_NS = 16
_NW = _NC * _NS
_CHUNK = 128
_CPW = 16


def _sc_gather(embedding, idx):
    epad = jnp.pad(embedding, ((0, 0), (0, _CHUNK - _D)))
    idx3 = idx.reshape(_NW, _CPW, _CHUNK)
    mesh = plsc.VectorSubcoreMesh(core_axis_name="c", subcore_axis_name="s")

    @functools.partial(
        pl.kernel,
        mesh=mesh,
        out_type=jax.ShapeDtypeStruct((_NW, _CPW, _CHUNK, _CHUNK), jnp.float32),
        scratch_types=[
            pltpu.VMEM((_CPW, _CHUNK), jnp.int32),
            pltpu.VMEM((4, _CHUNK, _CHUNK), jnp.float32),
            pltpu.SemaphoreType.DMA,
        ],
    )
    def gather_k(e_hbm, idx_hbm, out_hbm, idx_v, rows_v, sem):
        wid = lax.axis_index("c") * _NS + lax.axis_index("s")
        pltpu.sync_copy(idx_hbm.at[wid], idx_v)
        for grp in range(4):
            handles = []
            for b in range(4):
                j = grp * 4 + b
                handles.append(
                    pltpu.async_copy(e_hbm.at[idx_v.at[j]], rows_v.at[b], sem)
                )
            for h in handles:
                h.wait()
            for b in range(4):
                j = grp * 4 + b
                pltpu.sync_copy(rows_v.at[b], out_hbm.at[wid, j])

    out = gather_k(epad, idx3)
    return out.reshape(_NW * _CPW * _CHUNK, _CHUNK)[:, :_D]


# speedup vs baseline: 2.5566x; 2.5566x over previous
import jax
import jax.numpy as jnp
from jax.experimental import pallas as pl
from jax.experimental.pallas import tpu as pltpu

_K = 1024
_D = 64
_B = 1024
_COMMITMENT_COST = 1.0


_SL = 2


def _vq_tc(xt_ref, et_ref, qt_ref, idx_ref, loss_ref):
    et = et_ref[:]
    en = jnp.sum(et * et, axis=0)
    et_bf = et.astype(jnp.bfloat16)
    iota = jax.lax.broadcasted_iota(jnp.int32, (_K, _B), 0)
    i = pl.program_id(0)

    @pl.when(i == 0)
    def _init():
        loss_ref[0, 0] = 0.0

    for sl in range(_SL):
        xt = xt_ref[sl]
        prod = jax.lax.dot_general(
            et, xt, (((0,), (0,)), ((), ())),
            preferred_element_type=jnp.float32,
        )
        dist = en[:, None] - 2.0 * prod
        idx = jnp.argmin(dist, axis=0).astype(jnp.int32)
        idx_ref[pl.ds(sl * _B, _B)] = idx
        oh = (iota == idx[None, :]).astype(jnp.bfloat16)
        qt = jax.lax.dot_general(
            et_bf, oh, (((1,), (0,)), ((), ())),
            preferred_element_type=jnp.float32,
        )
        qt_ref[:, pl.ds(sl * _B, _B)] = qt
        loss_ref[0, 0] += jnp.sum((xt - qt) ** 2)


def kernel(inputs, embedding):
    g, bper, _ = inputs.shape
    n = g * bper
    xt3 = jnp.transpose(inputs, (0, 2, 1))
    et = embedding.T
    qt, idx, losssum = pl.pallas_call(
        _vq_tc,
        grid=(n // (_SL * _B),),
        in_specs=[
            pl.BlockSpec((_SL, _D, bper), lambda i: (i, 0, 0)),
            pl.BlockSpec((_D, _K), lambda i: (0, 0)),
        ],
        out_specs=[
            pl.BlockSpec((_D, _SL * _B), lambda i: (0, i)),
            pl.BlockSpec((_SL * _B,), lambda i: (i,)),
            pl.BlockSpec((1, 1), lambda i: (0, 0), memory_space=pltpu.SMEM),
        ],
        out_shape=[
            jax.ShapeDtypeStruct((_D, n), jnp.float32),
            jax.ShapeDtypeStruct((n,), jnp.int32),
            jax.ShapeDtypeStruct((1, 1), jnp.float32),
        ],
    )(xt3, et)
    loss = _COMMITMENT_COST * (losssum[0, 0] / (n * _D))
    return qt.T, loss, idx


# speedup vs baseline: 2.7247x; 1.0657x over previous
import jax
import jax.numpy as jnp
from jax.experimental import pallas as pl
from jax.experimental.pallas import tpu as pltpu

_K = 1024
_D = 64
_B = 1024
_COMMITMENT_COST = 1.0


_SL = 4


def _vq_tc(xt_ref, et_ref, qt_ref, idx_ref, loss_ref):
    et = et_ref[:]
    en = jnp.sum(et * et, axis=0)
    et_bf = et.astype(jnp.bfloat16)
    iota = jax.lax.broadcasted_iota(jnp.int32, (_K, _B), 0)
    i = pl.program_id(0)

    @pl.when(i == 0)
    def _init():
        loss_ref[0, 0] = 0.0

    for sl in range(_SL):
        xt = xt_ref[sl]
        prod = jax.lax.dot_general(
            et, xt, (((0,), (0,)), ((), ())),
            preferred_element_type=jnp.float32,
        )
        dist = en[:, None] - 2.0 * prod
        idx = jnp.argmin(dist, axis=0).astype(jnp.int32)
        idx_ref[pl.ds(sl * _B, _B)] = idx
        oh = (iota == idx[None, :]).astype(jnp.bfloat16)
        qt = jax.lax.dot_general(
            et_bf, oh, (((1,), (0,)), ((), ())),
            preferred_element_type=jnp.float32,
        )
        qt_ref[:, pl.ds(sl * _B, _B)] = qt
        loss_ref[0, 0] += jnp.sum((xt - qt) ** 2)


def kernel(inputs, embedding):
    g, bper, _ = inputs.shape
    n = g * bper
    xt3 = jnp.transpose(inputs, (0, 2, 1))
    et = embedding.T
    qt, idx, losssum = pl.pallas_call(
        _vq_tc,
        grid=(n // (_SL * _B),),
        in_specs=[
            pl.BlockSpec((_SL, _D, bper), lambda i: (i, 0, 0)),
            pl.BlockSpec((_D, _K), lambda i: (0, 0)),
        ],
        out_specs=[
            pl.BlockSpec((_D, _SL * _B), lambda i: (0, i)),
            pl.BlockSpec((_SL * _B,), lambda i: (i,)),
            pl.BlockSpec((1, 1), lambda i: (0, 0), memory_space=pltpu.SMEM),
        ],
        out_shape=[
            jax.ShapeDtypeStruct((_D, n), jnp.float32),
            jax.ShapeDtypeStruct((n,), jnp.int32),
            jax.ShapeDtypeStruct((1, 1), jnp.float32),
        ],
    )(xt3, et)
    loss = _COMMITMENT_COST * (losssum[0, 0] / (n * _D))
    return qt.T, loss, idx


# speedup vs baseline: 2.8299x; 1.0386x over previous
import jax
import jax.numpy as jnp
from jax.experimental import pallas as pl
from jax.experimental.pallas import tpu as pltpu

_K = 1024
_D = 64
_B = 1024
_COMMITMENT_COST = 1.0


_SL = 8


def _vq_tc(xt_ref, et_ref, qt_ref, idx_ref, loss_ref):
    et = et_ref[:]
    en = jnp.sum(et * et, axis=0)
    et_bf = et.astype(jnp.bfloat16)
    iota = jax.lax.broadcasted_iota(jnp.int32, (_K, _B), 0)
    i = pl.program_id(0)

    @pl.when(i == 0)
    def _init():
        loss_ref[0, 0] = 0.0

    for sl in range(_SL):
        xt = xt_ref[sl]
        prod = jax.lax.dot_general(
            et, xt, (((0,), (0,)), ((), ())),
            preferred_element_type=jnp.float32,
        )
        dist = en[:, None] - 2.0 * prod
        idx = jnp.argmin(dist, axis=0).astype(jnp.int32)
        idx_ref[pl.ds(sl * _B, _B)] = idx
        oh = (iota == idx[None, :]).astype(jnp.bfloat16)
        qt = jax.lax.dot_general(
            et_bf, oh, (((1,), (0,)), ((), ())),
            preferred_element_type=jnp.float32,
        )
        qt_ref[:, pl.ds(sl * _B, _B)] = qt
        loss_ref[0, 0] += jnp.sum((xt - qt) ** 2)


def kernel(inputs, embedding):
    g, bper, _ = inputs.shape
    n = g * bper
    xt3 = jnp.transpose(inputs, (0, 2, 1))
    et = embedding.T
    qt, idx, losssum = pl.pallas_call(
        _vq_tc,
        grid=(n // (_SL * _B),),
        in_specs=[
            pl.BlockSpec((_SL, _D, bper), lambda i: (i, 0, 0)),
            pl.BlockSpec((_D, _K), lambda i: (0, 0)),
        ],
        out_specs=[
            pl.BlockSpec((_D, _SL * _B), lambda i: (0, i)),
            pl.BlockSpec((_SL * _B,), lambda i: (i,)),
            pl.BlockSpec((1, 1), lambda i: (0, 0), memory_space=pltpu.SMEM),
        ],
        out_shape=[
            jax.ShapeDtypeStruct((_D, n), jnp.float32),
            jax.ShapeDtypeStruct((n,), jnp.int32),
            jax.ShapeDtypeStruct((1, 1), jnp.float32),
        ],
    )(xt3, et)
    loss = _COMMITMENT_COST * (losssum[0, 0] / (n * _D))
    return qt.T, loss, idx


# speedup vs baseline: 2.8525x; 1.0080x over previous
import jax
import jax.numpy as jnp
from jax.experimental import pallas as pl
from jax.experimental.pallas import tpu as pltpu

_K = 1024
_D = 64
_B = 1024
_COMMITMENT_COST = 1.0


_SL = 16


def _vq_tc(xt_ref, et_ref, qt_ref, idx_ref, loss_ref):
    et = et_ref[:]
    en = jnp.sum(et * et, axis=0)
    et_bf = et.astype(jnp.bfloat16)
    iota = jax.lax.broadcasted_iota(jnp.int32, (_K, _B), 0)
    i = pl.program_id(0)

    @pl.when(i == 0)
    def _init():
        loss_ref[0, 0] = 0.0

    for sl in range(_SL):
        xt = xt_ref[sl]
        prod = jax.lax.dot_general(
            et, xt, (((0,), (0,)), ((), ())),
            preferred_element_type=jnp.float32,
        )
        dist = en[:, None] - 2.0 * prod
        idx = jnp.argmin(dist, axis=0).astype(jnp.int32)
        idx_ref[pl.ds(sl * _B, _B)] = idx
        oh = (iota == idx[None, :]).astype(jnp.bfloat16)
        qt = jax.lax.dot_general(
            et_bf, oh, (((1,), (0,)), ((), ())),
            preferred_element_type=jnp.float32,
        )
        qt_ref[:, pl.ds(sl * _B, _B)] = qt
        loss_ref[0, 0] += jnp.sum((xt - qt) ** 2)


def kernel(inputs, embedding):
    g, bper, _ = inputs.shape
    n = g * bper
    xt3 = jnp.transpose(inputs, (0, 2, 1))
    et = embedding.T
    qt, idx, losssum = pl.pallas_call(
        _vq_tc,
        grid=(n // (_SL * _B),),
        in_specs=[
            pl.BlockSpec((_SL, _D, bper), lambda i: (i, 0, 0)),
            pl.BlockSpec((_D, _K), lambda i: (0, 0)),
        ],
        out_specs=[
            pl.BlockSpec((_D, _SL * _B), lambda i: (0, i)),
            pl.BlockSpec((_SL * _B,), lambda i: (i,)),
            pl.BlockSpec((1, 1), lambda i: (0, 0), memory_space=pltpu.SMEM),
        ],
        out_shape=[
            jax.ShapeDtypeStruct((_D, n), jnp.float32),
            jax.ShapeDtypeStruct((n,), jnp.int32),
            jax.ShapeDtypeStruct((1, 1), jnp.float32),
        ],
    )(xt3, et)
    loss = _COMMITMENT_COST * (losssum[0, 0] / (n * _D))
    return qt.T, loss, idx
